# Initial kernel scaffold; baseline (speedup 1.0000x reference)
import jax
import jax.numpy as jnp
from jax.experimental import pallas as pl


def kernel(x, W, b):
    raise NotImplementedError("write your pallas kernel here")


# speedup vs baseline: 1.0944x; 1.0944x over previous
import jax
import jax.numpy as jnp
from jax.experimental import pallas as pl

INPUT_DIM = 4096
NUM_EXPERTS = 64
TOP_K = 8
TOKENS = 16384
BLOCK_T = 512


def _gating_body(x_ref, w_ref, b_ref, probs_ref, topp_ref, topi_ref):
    x = x_ref[...]
    w = w_ref[...]
    logits = jax.lax.dot_general(
        x, w,
        dimension_numbers=(((1,), (1,)), ((), ())),
        preferred_element_type=jnp.float32,
    )
    logits = logits + b_ref[...]

    m = jnp.max(logits, axis=-1, keepdims=True)
    e = jnp.exp(logits - m)
    s = jnp.sum(e, axis=-1, keepdims=True)
    p = e / s
    probs_ref[...] = p

    lane = jax.lax.broadcasted_iota(jnp.int32, p.shape, 1)
    work = p
    vals = []
    idxs = []
    for _ in range(TOP_K):
        cur = jnp.max(work, axis=-1, keepdims=True)
        hit = work == cur
        idx = jnp.min(jnp.where(hit, lane, NUM_EXPERTS), axis=-1, keepdims=True)
        vals.append(cur)
        idxs.append(idx)
        work = jnp.where(lane == idx, -1.0, work)
    topv = jnp.concatenate(vals, axis=1)
    total = jnp.sum(topv, axis=-1, keepdims=True)
    topp_ref[...] = topv / total
    topi_ref[...] = jnp.concatenate(idxs, axis=1)


@jax.jit
def kernel(x, W, b):
    b2 = b.reshape(1, NUM_EXPERTS)
    probs, topp, topi = pl.pallas_call(
        _gating_body,
        grid=(TOKENS // BLOCK_T,),
        in_specs=[
            pl.BlockSpec((BLOCK_T, INPUT_DIM), lambda i: (i, 0)),
            pl.BlockSpec((NUM_EXPERTS, INPUT_DIM), lambda i: (0, 0)),
            pl.BlockSpec((1, NUM_EXPERTS), lambda i: (0, 0)),
        ],
        out_specs=[
            pl.BlockSpec((BLOCK_T, NUM_EXPERTS), lambda i: (i, 0)),
            pl.BlockSpec((BLOCK_T, TOP_K), lambda i: (i, 0)),
            pl.BlockSpec((BLOCK_T, TOP_K), lambda i: (i, 0)),
        ],
        out_shape=[
            jax.ShapeDtypeStruct((TOKENS, NUM_EXPERTS), jnp.float32),
            jax.ShapeDtypeStruct((TOKENS, TOP_K), jnp.float32),
            jax.ShapeDtypeStruct((TOKENS, TOP_K), jnp.int32),
        ],
    )(x, W, b2)
    return topp, topi, probs


# speedup vs baseline: 1.3183x; 1.2045x over previous
import jax
import jax.numpy as jnp
from jax.experimental import pallas as pl

INPUT_DIM = 4096
NUM_EXPERTS = 64
TOP_K = 8
TOKENS = 16384
BLOCK_T = 512


def _gating_body(x_ref, w_ref, b_ref, probs_ref, topp_ref, topi_ref):
    x = x_ref[...]
    w = w_ref[...]
    logits = jax.lax.dot_general(
        w, x,
        dimension_numbers=(((1,), (1,)), ((), ())),
        preferred_element_type=jnp.float32,
    )
    logits = logits + b_ref[...]

    m = jnp.max(logits, axis=0, keepdims=True)
    e = jnp.exp(logits - m)
    s = jnp.sum(e, axis=0, keepdims=True)
    probs_ref[...] = (e / s).T

    row = jax.lax.broadcasted_iota(jnp.int32, logits.shape, 0)
    work = logits
    vals = []
    idxs = []
    for _ in range(TOP_K):
        cur = jnp.max(work, axis=0, keepdims=True)
        hit = work == cur
        idx = jnp.min(jnp.where(hit, row, NUM_EXPERTS), axis=0, keepdims=True)
        vals.append(cur)
        idxs.append(idx)
        work = jnp.where(row == idx, -jnp.inf, work)
    topl = jnp.concatenate(vals, axis=0)
    topv = jnp.exp(topl - m) / s
    total = jnp.sum(topv, axis=0, keepdims=True)
    topp_ref[...] = (topv / total).T
    topi_ref[...] = jnp.concatenate(idxs, axis=0).T


@jax.jit
def kernel(x, W, b):
    b2 = b.reshape(NUM_EXPERTS, 1)
    probs, topp, topi = pl.pallas_call(
        _gating_body,
        grid=(TOKENS // BLOCK_T,),
        in_specs=[
            pl.BlockSpec((BLOCK_T, INPUT_DIM), lambda i: (i, 0)),
            pl.BlockSpec((NUM_EXPERTS, INPUT_DIM), lambda i: (0, 0)),
            pl.BlockSpec((NUM_EXPERTS, 1), lambda i: (0, 0)),
        ],
        out_specs=[
            pl.BlockSpec((BLOCK_T, NUM_EXPERTS), lambda i: (i, 0)),
            pl.BlockSpec((BLOCK_T, TOP_K), lambda i: (i, 0)),
            pl.BlockSpec((BLOCK_T, TOP_K), lambda i: (i, 0)),
        ],
        out_shape=[
            jax.ShapeDtypeStruct((TOKENS, NUM_EXPERTS), jnp.float32),
            jax.ShapeDtypeStruct((TOKENS, TOP_K), jnp.float32),
            jax.ShapeDtypeStruct((TOKENS, TOP_K), jnp.int32),
        ],
    )(x, W, b2)
    return topp, topi, probs


# speedup vs baseline: 1.4041x; 1.0652x over previous
import jax
import jax.numpy as jnp
from jax.experimental import pallas as pl

INPUT_DIM = 4096
NUM_EXPERTS = 64
TOP_K = 8
TOKENS = 16384
BLOCK_T = 1024


def _gating_body(x_ref, w_ref, b_ref, probs_ref, topp_ref, topi_ref):
    x = x_ref[...]
    w = w_ref[...]
    logits = jax.lax.dot_general(
        w, x,
        dimension_numbers=(((1,), (1,)), ((), ())),
        preferred_element_type=jnp.float32,
    )
    logits = logits + b_ref[...]

    m = jnp.max(logits, axis=0, keepdims=True)
    e = jnp.exp(logits - m)
    s = jnp.sum(e, axis=0, keepdims=True)
    probs_ref[...] = (e / s).T

    row = jax.lax.broadcasted_iota(jnp.int32, logits.shape, 0)
    work = logits
    vals = []
    idxs = []
    for _ in range(TOP_K):
        cur = jnp.max(work, axis=0, keepdims=True)
        hit = work == cur
        idx = jnp.min(jnp.where(hit, row, NUM_EXPERTS), axis=0, keepdims=True)
        vals.append(cur)
        idxs.append(idx)
        work = jnp.where(row == idx, -jnp.inf, work)
    topl = jnp.concatenate(vals, axis=0)
    topv = jnp.exp(topl - m) / s
    total = jnp.sum(topv, axis=0, keepdims=True)
    topp_ref[...] = (topv / total).T
    topi_ref[...] = jnp.concatenate(idxs, axis=0).T


@jax.jit
def kernel(x, W, b):
    b2 = b.reshape(NUM_EXPERTS, 1)
    probs, topp, topi = pl.pallas_call(
        _gating_body,
        grid=(TOKENS // BLOCK_T,),
        in_specs=[
            pl.BlockSpec((BLOCK_T, INPUT_DIM), lambda i: (i, 0)),
            pl.BlockSpec((NUM_EXPERTS, INPUT_DIM), lambda i: (0, 0)),
            pl.BlockSpec((NUM_EXPERTS, 1), lambda i: (0, 0)),
        ],
        out_specs=[
            pl.BlockSpec((BLOCK_T, NUM_EXPERTS), lambda i: (i, 0)),
            pl.BlockSpec((BLOCK_T, TOP_K), lambda i: (i, 0)),
            pl.BlockSpec((BLOCK_T, TOP_K), lambda i: (i, 0)),
        ],
        out_shape=[
            jax.ShapeDtypeStruct((TOKENS, NUM_EXPERTS), jnp.float32),
            jax.ShapeDtypeStruct((TOKENS, TOP_K), jnp.float32),
            jax.ShapeDtypeStruct((TOKENS, TOP_K), jnp.int32),
        ],
    )(x, W, b2)
    return topp, topi, probs
